# baseline (speedup 1.0000x reference)
import functools

import jax
import jax.numpy as jnp
from jax import lax
from jax.experimental import pallas as pl
from jax.experimental.pallas import tpu as pltpu
from jax.experimental.pallas import tpu_sc as plsc

V = 100000
CH = 24
SEG = 6400
TILE_ROWS = V // 8


def _sc_geometry():
    try:
        info = plsc.get_sparse_core_info()
        return info.num_cores, info.num_subcores
    except Exception:
        return 2, 16


@functools.lru_cache(maxsize=None)
def _build(B, L):
    NC, NS = _sc_geometry()
    NW = NC * NS
    NE = B * L
    n_seg = NE // SEG
    n_ic = SEG // 16
    rows_min = 8 * ((TILE_ROWS * 1) // NW) * 1
    n_chunks = (8 * (TILE_ROWS // NW)) // CH
    assert SEG % L == 0 and NE % SEG == 0
    assert (8 * (TILE_ROWS // NW)) % CH == 0
    SENTINEL = jnp.int32(4095 << 10)

    mesh = plsc.VectorSubcoreMesh(core_axis_name="c", subcore_axis_name="s")

    @functools.partial(
        pl.kernel,
        out_type=jax.ShapeDtypeStruct((V, B), jnp.float32),
        mesh=mesh,
        compiler_params=pltpu.CompilerParams(needs_layout_passes=False),
        scratch_types=[
            pltpu.VMEM((SEG,), jnp.int32),
            pltpu.VMEM((SEG,), jnp.int32),
            pltpu.VMEM((NE + 16,), jnp.int32),
            pltpu.VMEM((CH, B), jnp.float32),
            pltpu.VMEM((CH, B), jnp.float32),
            pltpu.SemaphoreType.DMA,
            pltpu.SemaphoreType.DMA,
            pltpu.SemaphoreType.DMA,
            pltpu.SemaphoreType.DMA,
        ],
    )
    def harden(vec_hbm, val_hbm, out_hbm, vsegA, vsegB, plist, buf0, buf1,
               sem0, sem1, ssemA, ssemB):
        wid = lax.axis_index("s") * NC + lax.axis_index("c")
        t0 = (TILE_ROWS * wid) >> 5
        t1 = (TILE_ROWS * (wid + 1)) >> 5
        vbase = t0 * 8
        nrows = (t1 - t0) * 8

        lane = lax.iota(jnp.int32, 16)
        zeros16 = jnp.zeros((16,), jnp.float32)
        ones16 = jnp.ones((16,), jnp.float32)

        pltpu.async_copy(vec_hbm.at[pl.ds(0, SEG)], vsegA, ssemA)

        for buf in (buf0, buf1):
            def _z(i, carry, buf=buf):
                buf[i >> 6, pl.ds((i & 63) * 16, 16)] = zeros16
                return carry
            lax.fori_loop(0, CH * B // 16, _z, 0)

        def scan_seg(vseg, seg, cnt):
            row0 = seg * (SEG // L)

            def ic_body(ic, cnt):
                v16 = vseg[pl.ds(ic * 16, 16)]
                q16 = lane + ic * 16
                r16 = row0 + ((q16 * 20972) >> 20)
                t16 = v16 - vbase
                mask = (t16 >= 0) & (t16 < nrows)
                packed = lax.shift_left(t16, 10) | r16
                plsc.store_compressed(plist.at[pl.ds(cnt, 16)], packed,
                                      mask=mask)
                npick = lax.reduce_max(
                    plsc.all_reduce_population_count(mask), axes=(0,))
                return cnt + npick

            return lax.fori_loop(0, n_ic, ic_body, cnt)

        def seg_pair(gp, cnt):
            segA = gp * 2
            pltpu.make_async_copy(vec_hbm.at[pl.ds(0, SEG)], vsegA,
                                  ssemA).wait()
            pltpu.async_copy(
                vec_hbm.at[pl.ds((segA + 1) * SEG, SEG)], vsegB, ssemB)
            cnt = scan_seg(vsegA, segA, cnt)
            pltpu.make_async_copy(vec_hbm.at[pl.ds(0, SEG)], vsegB,
                                  ssemB).wait()

            @pl.when(gp < n_seg // 2 - 1)
            def _next():
                pltpu.async_copy(
                    vec_hbm.at[pl.ds((segA + 2) * SEG, SEG)], vsegA, ssemA)

            return scan_seg(vsegB, segA + 1, cnt)

        cnt = lax.fori_loop(0, n_seg // 2, seg_pair, jnp.int32(0))
        plist[pl.ds(cnt, 16)] = jnp.full((16,), SENTINEL, jnp.int32)
        trips = (cnt + 15) >> 4

        def scan(buf, base, x16):
            wb = lax.shift_left(base, 10)
            span = CH << 10

            def body(i, carry):
                p16 = plist[pl.ds(i * 16, 16)]
                dw = p16 - wb
                mw = (dw >= 0) & (dw < span)
                plsc.store_scatter(
                    buf, [lax.shift_right_logical(dw, 10), p16 & 1023],
                    x16, mask=mw)
                return carry

            lax.fori_loop(0, trips, body, 0)

        def drain(buf, sem):
            pltpu.make_async_copy(buf, out_hbm.at[pl.ds(0, CH)], sem).wait()

        def step(g, carry):
            for k, (buf, sem) in enumerate(((buf0, sem0), (buf1, sem1))):
                cc = g * 2 + k

                @pl.when(g >= 1)
                def _w(cc=cc, buf=buf, sem=sem):
                    drain(buf, sem)
                    scan(buf, (cc - 2) * CH, zeros16)

                scan(buf, cc * CH, ones16)
                pltpu.async_copy(
                    buf, out_hbm.at[pl.ds(vbase + cc * CH, CH)], sem)
            return carry

        lax.fori_loop(0, n_chunks // 2, step, 0)

        drain(buf0, sem0)
        drain(buf1, sem1)

        @pl.when(nrows > n_chunks * CH)
        def _tail():
            scan(buf0, (n_chunks - 2) * CH, zeros16)
            scan(buf0, n_chunks * CH, ones16)
            pltpu.async_copy(
                buf0.at[pl.ds(0, 8)],
                out_hbm.at[pl.ds(vbase + n_chunks * CH, 8)], sem0).wait()

    return harden


def kernel(vec, val):
    B, L = vec.shape
    harden = _build(B, L)
    return harden(vec.reshape(-1), val.reshape(-1)).T


# speedup vs baseline: 1.3794x; 1.3794x over previous
import functools

import jax
import jax.numpy as jnp
from jax import lax
from jax.experimental import pallas as pl
from jax.experimental.pallas import tpu as pltpu
from jax.experimental.pallas import tpu_sc as plsc

V = 100000
CH = 24
SEG = 6400
TILE_ROWS = V // 8


def _sc_geometry():
    try:
        info = plsc.get_sparse_core_info()
        return info.num_cores, info.num_subcores
    except Exception:
        return 2, 16


@functools.lru_cache(maxsize=None)
def _build(B, L):
    NC, NS = _sc_geometry()
    NW = NC * NS
    NE = B * L
    n_seg = NE // SEG
    n_ic = SEG // 16
    rows_min = 8 * ((TILE_ROWS * 1) // NW) * 1
    n_chunks = (8 * (TILE_ROWS // NW)) // CH
    assert SEG % L == 0 and NE % SEG == 0
    assert (8 * (TILE_ROWS // NW)) % CH == 0
    SENTINEL = jnp.int32(4095 << 10)

    mesh = plsc.VectorSubcoreMesh(core_axis_name="c", subcore_axis_name="s")

    @functools.partial(
        pl.kernel,
        out_type=jax.ShapeDtypeStruct((V, B), jnp.float32),
        mesh=mesh,
        compiler_params=pltpu.CompilerParams(needs_layout_passes=False),
        scratch_types=[
            pltpu.VMEM((SEG,), jnp.int32),
            pltpu.VMEM((SEG,), jnp.int32),
            pltpu.VMEM((NE + 16,), jnp.int32),
            pltpu.VMEM((CH, B), jnp.float32),
            pltpu.VMEM((CH, B), jnp.float32),
            pltpu.VMEM((2080,), jnp.int32),
            pltpu.VMEM((2080,), jnp.int32),
            pltpu.SemaphoreType.DMA,
            pltpu.SemaphoreType.DMA,
            pltpu.SemaphoreType.DMA,
            pltpu.SemaphoreType.DMA,
        ],
    )
    def harden(vec_hbm, val_hbm, out_hbm, vsegA, vsegB, plist, buf0, buf1,
               tch0, tch1, sem0, sem1, ssemA, ssemB):
        wid = lax.axis_index("s") * NC + lax.axis_index("c")
        t0 = (TILE_ROWS * wid) >> 5
        t1 = (TILE_ROWS * (wid + 1)) >> 5
        vbase = t0 * 8
        nrows = (t1 - t0) * 8

        lane = lax.iota(jnp.int32, 16)
        zeros16 = jnp.zeros((16,), jnp.float32)
        ones16 = jnp.ones((16,), jnp.float32)

        pltpu.async_copy(vec_hbm.at[pl.ds(0, SEG)], vsegA, ssemA)

        for buf in (buf0, buf1):
            def _z(i, carry, buf=buf):
                buf[i >> 6, pl.ds((i & 63) * 16, 16)] = zeros16
                return carry
            lax.fori_loop(0, CH * B // 16, _z, 0)

        def scan_seg(vseg, seg, cnt):
            row0 = seg * (SEG // L)

            def ic_body(ic, cnt):
                v16 = vseg[pl.ds(ic * 16, 16)]
                q16 = lane + ic * 16
                r16 = row0 + ((q16 * 20972) >> 20)
                t16 = v16 - vbase
                mask = (t16 >= 0) & (t16 < nrows)
                packed = lax.shift_left(t16, 10) | r16
                plsc.store_compressed(plist.at[pl.ds(cnt, 16)], packed,
                                      mask=mask)
                npick = lax.reduce_max(
                    plsc.all_reduce_population_count(mask), axes=(0,))
                return cnt + npick

            return lax.fori_loop(0, n_ic, ic_body, cnt)

        def seg_pair(gp, cnt):
            segA = gp * 2
            pltpu.make_async_copy(vec_hbm.at[pl.ds(0, SEG)], vsegA,
                                  ssemA).wait()
            pltpu.async_copy(
                vec_hbm.at[pl.ds((segA + 1) * SEG, SEG)], vsegB, ssemB)
            cnt = scan_seg(vsegA, segA, cnt)
            pltpu.make_async_copy(vec_hbm.at[pl.ds(0, SEG)], vsegB,
                                  ssemB).wait()

            @pl.when(gp < n_seg // 2 - 1)
            def _next():
                pltpu.async_copy(
                    vec_hbm.at[pl.ds((segA + 2) * SEG, SEG)], vsegA, ssemA)

            return scan_seg(vsegB, segA + 1, cnt)

        cnt = lax.fori_loop(0, n_seg // 2, seg_pair, jnp.int32(0))
        plist[pl.ds(cnt, 16)] = jnp.full((16,), SENTINEL, jnp.int32)
        trips = (cnt + 15) >> 4

        CAP = 2048

        def scan(buf, base, x16, tch=None):
            wb = lax.shift_left(base, 10)
            span = CH << 10

            def body(i, tcnt):
                p16 = plist[pl.ds(i * 16, 16)]
                dw = p16 - wb
                mw = (dw >= 0) & (dw < span)
                plsc.store_scatter(
                    buf, [lax.shift_right_logical(dw, 10), p16 & 1023],
                    x16, mask=mw)
                if tch is None:
                    return tcnt
                plsc.store_compressed(
                    tch.at[pl.ds(jnp.minimum(tcnt, CAP), 16)], dw, mask=mw)
                return tcnt + lax.reduce_max(
                    plsc.all_reduce_population_count(mw), axes=(0,))

            return lax.fori_loop(0, trips, body, jnp.int32(0))

        def restore(buf, base, tch, tcnt):
            @pl.when(tcnt <= CAP)
            def _fast():
                def body(i, carry):
                    c16 = tch[pl.ds(i * 16, 16)]
                    m = (lane + i * 16) < tcnt
                    plsc.store_scatter(
                        buf,
                        [lax.shift_right_logical(c16, 10), c16 & 1023],
                        zeros16, mask=m)
                    return carry
                lax.fori_loop(0, (tcnt + 15) >> 4, body, 0)

            @pl.when(tcnt > CAP)
            def _slow():
                scan(buf, base, zeros16)

        def drain(buf, sem):
            pltpu.make_async_copy(buf, out_hbm.at[pl.ds(0, CH)], sem).wait()

        def step(g, carry):
            tcs = list(carry)
            for k, (buf, sem, tch) in enumerate(
                    ((buf0, sem0, tch0), (buf1, sem1, tch1))):
                cc = g * 2 + k

                @pl.when(g >= 1)
                def _w(cc=cc, buf=buf, sem=sem, tch=tch, tc=tcs[k]):
                    drain(buf, sem)
                    restore(buf, (cc - 2) * CH, tch, tc)

                tcs[k] = scan(buf, cc * CH, ones16, tch=tch)
                pltpu.async_copy(
                    buf, out_hbm.at[pl.ds(vbase + cc * CH, CH)], sem)
            return tuple(tcs)

        tc0, tc1 = lax.fori_loop(0, n_chunks // 2, step,
                                 (jnp.int32(0), jnp.int32(0)))

        drain(buf0, sem0)
        drain(buf1, sem1)

        @pl.when(nrows > n_chunks * CH)
        def _tail():
            restore(buf0, (n_chunks - 2) * CH, tch0, tc0)
            scan(buf0, n_chunks * CH, ones16)
            pltpu.async_copy(
                buf0.at[pl.ds(0, 8)],
                out_hbm.at[pl.ds(vbase + n_chunks * CH, 8)], sem0).wait()

    return harden


def kernel(vec, val):
    B, L = vec.shape
    harden = _build(B, L)
    return harden(vec.reshape(-1), val.reshape(-1)).T


# speedup vs baseline: 1.4004x; 1.0153x over previous
import functools

import jax
import jax.numpy as jnp
from jax import lax
from jax.experimental import pallas as pl
from jax.experimental.pallas import tpu as pltpu
from jax.experimental.pallas import tpu_sc as plsc

V = 100000
CH = 24
SEG = 6400
TILE_ROWS = V // 8


def _sc_geometry():
    try:
        info = plsc.get_sparse_core_info()
        return info.num_cores, info.num_subcores
    except Exception:
        return 2, 16


@functools.lru_cache(maxsize=None)
def _build(B, L):
    NC, NS = _sc_geometry()
    NW = NC * NS
    NE = B * L
    n_seg = NE // SEG
    n_ic = SEG // 16
    rows_min = 8 * ((TILE_ROWS * 1) // NW) * 1
    n_chunks = (8 * (TILE_ROWS // NW)) // CH
    assert SEG % L == 0 and NE % SEG == 0
    assert (8 * (TILE_ROWS // NW)) % CH == 0
    SENTINEL = jnp.int32(4095 << 10)

    mesh = plsc.VectorSubcoreMesh(core_axis_name="c", subcore_axis_name="s")

    @functools.partial(
        pl.kernel,
        out_type=jax.ShapeDtypeStruct((V, B), jnp.float32),
        mesh=mesh,
        compiler_params=pltpu.CompilerParams(needs_layout_passes=False),
        scratch_types=[
            pltpu.VMEM((SEG,), jnp.int32),
            pltpu.VMEM((SEG,), jnp.int32),
            pltpu.VMEM((NE + 16,), jnp.int32),
            pltpu.VMEM((CH, B), jnp.float32),
            pltpu.VMEM((CH, B), jnp.float32),
            pltpu.VMEM((2080,), jnp.int32),
            pltpu.VMEM((2080,), jnp.int32),
            pltpu.SemaphoreType.DMA,
            pltpu.SemaphoreType.DMA,
            pltpu.SemaphoreType.DMA,
            pltpu.SemaphoreType.DMA,
        ],
    )
    def harden(vec_hbm, val_hbm, out_hbm, vsegA, vsegB, plist, buf0, buf1,
               tch0, tch1, sem0, sem1, ssemA, ssemB):
        wid = lax.axis_index("s") * NC + lax.axis_index("c")
        t0 = (TILE_ROWS * wid) >> 5
        t1 = (TILE_ROWS * (wid + 1)) >> 5
        vbase = t0 * 8
        nrows = (t1 - t0) * 8

        lane = lax.iota(jnp.int32, 16)
        zeros16 = jnp.zeros((16,), jnp.float32)
        ones16 = jnp.ones((16,), jnp.float32)

        pltpu.async_copy(vec_hbm.at[pl.ds(0, SEG)], vsegA, ssemA)

        for buf in (buf0, buf1):
            def _z(i, carry, buf=buf):
                for u in range(4):
                    buf[i >> 4, pl.ds((i & 15) * 64 + u * 16, 16)] = zeros16
                return carry
            lax.fori_loop(0, CH * B // 64, _z, 0)

        def scan_seg(vseg, seg, cnt):
            row0 = seg * (SEG // L)

            def ic_half(ic, cnt):
                v16 = vseg[pl.ds(ic * 16, 16)]
                q16 = lane + ic * 16
                r16 = row0 + ((q16 * 20972) >> 20)
                t16 = v16 - vbase
                mask = (t16 >= 0) & (t16 < nrows)
                packed = lax.shift_left(t16, 10) | r16
                plsc.store_compressed(plist.at[pl.ds(cnt, 16)], packed,
                                      mask=mask)
                npick = lax.reduce_max(
                    plsc.all_reduce_population_count(mask), axes=(0,))
                return cnt + npick

            def ic_body(ic2, cnt):
                return ic_half(ic2 * 2 + 1, ic_half(ic2 * 2, cnt))

            return lax.fori_loop(0, n_ic // 2, ic_body, cnt)

        def seg_pair(gp, cnt):
            segA = gp * 2
            pltpu.make_async_copy(vec_hbm.at[pl.ds(0, SEG)], vsegA,
                                  ssemA).wait()
            pltpu.async_copy(
                vec_hbm.at[pl.ds((segA + 1) * SEG, SEG)], vsegB, ssemB)
            cnt = scan_seg(vsegA, segA, cnt)
            pltpu.make_async_copy(vec_hbm.at[pl.ds(0, SEG)], vsegB,
                                  ssemB).wait()

            @pl.when(gp < n_seg // 2 - 1)
            def _next():
                pltpu.async_copy(
                    vec_hbm.at[pl.ds((segA + 2) * SEG, SEG)], vsegA, ssemA)

            return scan_seg(vsegB, segA + 1, cnt)

        cnt = lax.fori_loop(0, n_seg // 2, seg_pair, jnp.int32(0))
        plist[pl.ds(cnt, 16)] = jnp.full((16,), SENTINEL, jnp.int32)
        trips = (cnt + 15) >> 4

        CAP = 2048

        def scan(buf, base, x16, tch=None):
            wb = lax.shift_left(base, 10)
            span = CH << 10

            def body(i, tcnt):
                p16 = plist[pl.ds(i * 16, 16)]
                dw = p16 - wb
                mw = (dw >= 0) & (dw < span)
                plsc.store_scatter(
                    buf, [lax.shift_right_logical(dw, 10), p16 & 1023],
                    x16, mask=mw)
                if tch is None:
                    return tcnt
                plsc.store_compressed(
                    tch.at[pl.ds(jnp.minimum(tcnt, CAP), 16)], dw, mask=mw)
                return tcnt + lax.reduce_max(
                    plsc.all_reduce_population_count(mw), axes=(0,))

            return lax.fori_loop(0, trips, body, jnp.int32(0))

        def restore(buf, base, tch, tcnt):
            @pl.when(tcnt <= CAP)
            def _fast():
                def body(i, carry):
                    c16 = tch[pl.ds(i * 16, 16)]
                    m = (lane + i * 16) < tcnt
                    plsc.store_scatter(
                        buf,
                        [lax.shift_right_logical(c16, 10), c16 & 1023],
                        zeros16, mask=m)
                    return carry
                lax.fori_loop(0, (tcnt + 15) >> 4, body, 0)

            @pl.when(tcnt > CAP)
            def _slow():
                scan(buf, base, zeros16)

        def drain(buf, sem):
            pltpu.make_async_copy(buf, out_hbm.at[pl.ds(0, CH)], sem).wait()

        def step(g, carry):
            tcs = list(carry)
            for k, (buf, sem, tch) in enumerate(
                    ((buf0, sem0, tch0), (buf1, sem1, tch1))):
                cc = g * 2 + k

                @pl.when(g >= 1)
                def _w(cc=cc, buf=buf, sem=sem, tch=tch, tc=tcs[k]):
                    drain(buf, sem)
                    restore(buf, (cc - 2) * CH, tch, tc)

                tcs[k] = scan(buf, cc * CH, ones16, tch=tch)
                pltpu.async_copy(
                    buf, out_hbm.at[pl.ds(vbase + cc * CH, CH)], sem)
            return tuple(tcs)

        tc0, tc1 = lax.fori_loop(0, n_chunks // 2, step,
                                 (jnp.int32(0), jnp.int32(0)))

        drain(buf0, sem0)
        drain(buf1, sem1)

        @pl.when(nrows > n_chunks * CH)
        def _tail():
            restore(buf0, (n_chunks - 2) * CH, tch0, tc0)
            scan(buf0, n_chunks * CH, ones16)
            pltpu.async_copy(
                buf0.at[pl.ds(0, 8)],
                out_hbm.at[pl.ds(vbase + n_chunks * CH, 8)], sem0).wait()

    return harden


def kernel(vec, val):
    B, L = vec.shape
    harden = _build(B, L)
    return harden(vec.reshape(-1), val.reshape(-1)).T
